# Initial kernel scaffold; baseline (speedup 1.0000x reference)
import functools

import jax
import jax.numpy as jnp
from jax import lax
from jax.experimental import pallas as pl
from jax.experimental.pallas import tpu as pltpu
from jax.experimental.pallas import tpu_sc as plsc

N = 10000
D_IN = 128
D_OUT = 200
E = 320000

NC = 2
NS = 16
NW = NC * NS

CHUNK = 128
CHUNKS_PER_W = 80
E_PAD = NW * CHUNKS_PER_W * CHUNK
ROWS_PER_TILE = N // NS

_mesh = plsc.VectorSubcoreMesh(
    core_axis_name="c", subcore_axis_name="s", num_cores=NC, num_subcores=NS
)


@functools.partial(
    pl.kernel,
    out_type=jax.ShapeDtypeStruct((NC, N, D_IN), jnp.float32),
    mesh=_mesh,
    scratch_types=[
        pltpu.VMEM((CHUNKS_PER_W, CHUNK), jnp.int32),
        pltpu.VMEM((CHUNKS_PER_W, CHUNK), jnp.int32),
        pltpu.VMEM((CHUNKS_PER_W, CHUNK), jnp.float32),
        pltpu.VMEM((CHUNK, D_IN), jnp.float32),
        pltpu.VMEM_SHARED((N, D_IN), jnp.float32),
        pltpu.SemaphoreType.DMA,
    ],
)
def _sc_message_pass(x_hbm, src_hbm, dst_hbm, w_hbm, zeros_hbm, part_hbm,
                     src_v, dst_v, w_v, rows_v, acc_s, sem):
    c = lax.axis_index("c")
    s = lax.axis_index("s")
    wid = c * NS + s

    pltpu.sync_copy(src_hbm.at[wid], src_v)
    pltpu.sync_copy(dst_hbm.at[wid], dst_v)
    pltpu.sync_copy(w_hbm.at[wid], w_v)

    stripe = pl.ds(s * ROWS_PER_TILE, ROWS_PER_TILE)
    pltpu.sync_copy(zeros_hbm.at[stripe], acc_s.at[stripe])
    plsc.subcore_barrier()

    def chunk_body(k, carry):
        pltpu.async_copy(x_hbm.at[src_v.at[k]], rows_v, sem).wait()

        def row_body(i, carry2):
            w = w_v[k, i]
            for j in range(D_IN // 16):
                sl = pl.ds(j * 16, 16)
                rows_v[i, sl] = rows_v[i, sl] * w
            return carry2

        lax.fori_loop(0, CHUNK, row_body, 0)

        pltpu.sync_copy(rows_v, acc_s.at[dst_v.at[k]], add=True)
        return carry

    lax.fori_loop(0, CHUNKS_PER_W, chunk_body, 0)
    plsc.subcore_barrier()

    pltpu.sync_copy(acc_s.at[stripe], part_hbm.at[c, stripe])


R_BLK = 1000


def _dense_body(p0_ref, p1_ref, x_ref, wrel_ref, brel_ref, wroot_ref,
                wdec_ref, bdec_ref, out_ref):
    agg = p0_ref[...] + p1_ref[...]
    z = (jnp.dot(agg, wrel_ref[...], preferred_element_type=jnp.float32)
         + jnp.dot(x_ref[...], wroot_ref[...], preferred_element_type=jnp.float32)
         + brel_ref[...])
    z = jnp.maximum(z, 0.0)
    out_ref[...] = (jnp.dot(z, wdec_ref[...], preferred_element_type=jnp.float32)
                    + bdec_ref[...])


def _dense(p0, p1, x, W_rel, b_rel2, W_root, W_dec, b_dec2):
    row = lambda r, cc: pl.BlockSpec((r, cc), lambda i: (i, 0))
    full = lambda a, b: pl.BlockSpec((a, b), lambda i: (0, 0))
    return pl.pallas_call(
        _dense_body,
        grid=(N // R_BLK,),
        in_specs=[row(R_BLK, D_IN), row(R_BLK, D_IN), row(R_BLK, D_IN),
                  full(D_IN, D_OUT), full(1, D_OUT), full(D_IN, D_OUT),
                  full(D_OUT, D_IN), full(1, D_IN)],
        out_specs=row(R_BLK, D_IN),
        out_shape=jax.ShapeDtypeStruct((N, D_IN), jnp.float32),
    )(p0, p1, x, W_rel, b_rel2, W_root, W_dec, b_dec2)


def kernel(x, edge_index, edge_weight, W_rel, b_rel, W_root, W_dec, b_dec):
    src = edge_index[0]
    dst = edge_index[1]
    pad = E_PAD - E
    src_r = jnp.pad(src, (0, pad)).reshape(NW, CHUNKS_PER_W, CHUNK)
    dst_r = jnp.pad(dst, (0, pad)).reshape(NW, CHUNKS_PER_W, CHUNK)
    w_r = jnp.pad(edge_weight, (0, pad)).reshape(NW, CHUNKS_PER_W, CHUNK)
    zeros = jnp.zeros((N, D_IN), jnp.float32)

    parts = _sc_message_pass(x, src_r, dst_r, w_r, zeros)
    return _dense(parts[0], parts[1], x, W_rel, b_rel.reshape(1, D_OUT),
                  W_root, W_dec, b_dec.reshape(1, D_IN))


# speedup vs baseline: 3.2309x; 3.2309x over previous
import functools

import jax
import jax.numpy as jnp
from jax import lax
from jax.experimental import pallas as pl
from jax.experimental.pallas import tpu as pltpu
from jax.experimental.pallas import tpu_sc as plsc

N = 10000
D_IN = 128
D_OUT = 200
E = 320000

NC = 2
NS = 16
NW = NC * NS

CHUNK = 128
CHUNKS_PER_W = 80
E_PAD = NW * CHUNKS_PER_W * CHUNK
N_PAD = 10240
ROWS_PER_TILE = N_PAD // NS

_mesh = plsc.VectorSubcoreMesh(
    core_axis_name="c", subcore_axis_name="s", num_cores=NC, num_subcores=NS
)


@functools.partial(
    pl.kernel,
    out_type=jax.ShapeDtypeStruct((NC, N_PAD, D_IN), jnp.float32),
    mesh=_mesh,
    scratch_types=[
        pltpu.VMEM((CHUNKS_PER_W, CHUNK), jnp.int32),
        pltpu.VMEM((CHUNKS_PER_W, CHUNK), jnp.int32),
        pltpu.VMEM((CHUNKS_PER_W, CHUNK), jnp.float32),
        pltpu.VMEM((CHUNK, D_IN), jnp.float32),
        pltpu.VMEM_SHARED((N_PAD, D_IN), jnp.float32),
        pltpu.SemaphoreType.DMA,
    ],
)
def _sc_message_pass(x_hbm, src_hbm, dst_hbm, w_hbm, zeros_hbm, part_hbm,
                     src_v, dst_v, w_v, rows_v, acc_s, sem):
    c = lax.axis_index("c")
    s = lax.axis_index("s")
    wid = c * NS + s

    pltpu.sync_copy(src_hbm.at[wid], src_v)
    pltpu.sync_copy(dst_hbm.at[wid], dst_v)
    pltpu.sync_copy(w_hbm.at[wid], w_v)

    stripe = pl.ds(s * ROWS_PER_TILE, ROWS_PER_TILE)
    pltpu.sync_copy(zeros_hbm.at[stripe], acc_s.at[stripe])
    plsc.subcore_barrier()

    def chunk_body(k, carry):
        pltpu.async_copy(x_hbm.at[src_v.at[k]], rows_v, sem).wait()

        def group_body(g, carry2):
            wv = w_v[k, pl.ds(g * 16, 16)]
            for t in range(16):
                i = g * 16 + t
                w = wv[t]
                for j in range(D_IN // 16):
                    sl = pl.ds(j * 16, 16)
                    rows_v[i, sl] = rows_v[i, sl] * w
            return carry2

        lax.fori_loop(0, CHUNK // 16, group_body, 0)

        pltpu.sync_copy(rows_v, acc_s.at[dst_v.at[k]], add=True)
        return carry

    lax.fori_loop(0, CHUNKS_PER_W, chunk_body, 0)
    plsc.subcore_barrier()

    pltpu.sync_copy(acc_s.at[stripe], part_hbm.at[c, stripe])


R_BLK = 1000


def _dense_body(p0_ref, p1_ref, x_ref, wrel_ref, brel_ref, wroot_ref,
                wdec_ref, bdec_ref, out_ref):
    agg = p0_ref[...] + p1_ref[...]
    z = (jnp.dot(agg, wrel_ref[...], preferred_element_type=jnp.float32)
         + jnp.dot(x_ref[...], wroot_ref[...], preferred_element_type=jnp.float32)
         + brel_ref[...])
    z = jnp.maximum(z, 0.0)
    out_ref[...] = (jnp.dot(z, wdec_ref[...], preferred_element_type=jnp.float32)
                    + bdec_ref[...])


def _dense(p0, p1, x, W_rel, b_rel2, W_root, W_dec, b_dec2):
    row = lambda r, cc: pl.BlockSpec((r, cc), lambda i: (i, 0))
    full = lambda a, b: pl.BlockSpec((a, b), lambda i: (0, 0))
    return pl.pallas_call(
        _dense_body,
        grid=(N // R_BLK,),
        in_specs=[row(R_BLK, D_IN), row(R_BLK, D_IN), row(R_BLK, D_IN),
                  full(D_IN, D_OUT), full(1, D_OUT), full(D_IN, D_OUT),
                  full(D_OUT, D_IN), full(1, D_IN)],
        out_specs=row(R_BLK, D_IN),
        out_shape=jax.ShapeDtypeStruct((N, D_IN), jnp.float32),
    )(p0, p1, x, W_rel, b_rel2, W_root, W_dec, b_dec2)


def kernel(x, edge_index, edge_weight, W_rel, b_rel, W_root, W_dec, b_dec):
    src = edge_index[0]
    dst = edge_index[1]
    pad = E_PAD - E
    src_r = jnp.pad(src, (0, pad)).reshape(NW, CHUNKS_PER_W, CHUNK)
    dst_r = jnp.pad(dst, (0, pad)).reshape(NW, CHUNKS_PER_W, CHUNK)
    w_r = jnp.pad(edge_weight, (0, pad)).reshape(NW, CHUNKS_PER_W, CHUNK)
    zeros = jnp.zeros((N_PAD, D_IN), jnp.float32)

    parts = _sc_message_pass(x, src_r, dst_r, w_r, zeros)
    return _dense(parts[0, :N], parts[1, :N], x, W_rel, b_rel.reshape(1, D_OUT),
                  W_root, W_dec, b_dec.reshape(1, D_IN))


# speedup vs baseline: 3.6749x; 1.1374x over previous
import functools

import jax
import jax.numpy as jnp
from jax import lax
from jax.experimental import pallas as pl
from jax.experimental.pallas import tpu as pltpu
from jax.experimental.pallas import tpu_sc as plsc

N = 10000
D_IN = 128
D_OUT = 200
E = 320000

NC = 2
NS = 16
NW = NC * NS

CHUNK = 128
CHUNKS_PER_W = 80
NQ = 4
H = CHUNKS_PER_W // NQ
NBUF = 2
E_PAD = NW * CHUNKS_PER_W * CHUNK
N_PAD = 10240
ROWS_PER_TILE = N_PAD // NS

_mesh = plsc.VectorSubcoreMesh(
    core_axis_name="c", subcore_axis_name="s", num_cores=NC, num_subcores=NS
)


@functools.partial(
    pl.kernel,
    out_type=jax.ShapeDtypeStruct((NC, N_PAD, D_IN), jnp.float32),
    mesh=_mesh,
    scratch_types=[
        pltpu.VMEM((H, CHUNK), jnp.int32),
        pltpu.VMEM((H, CHUNK), jnp.int32),
        pltpu.VMEM((H, CHUNK), jnp.float32),
        pltpu.VMEM((NBUF, CHUNK, D_IN), jnp.float32),
        pltpu.VMEM_SHARED((N_PAD, D_IN), jnp.float32),
        pltpu.SemaphoreType.DMA,
        pltpu.SemaphoreType.DMA,
        pltpu.SemaphoreType.DMA,
        pltpu.SemaphoreType.DMA,
    ],
)
def _sc_message_pass(x_hbm, src_hbm, dst_hbm, w_hbm, zeros_hbm, part_hbm,
                     src_v, dst_v, w_v, rows_v, acc_s,
                     g0, g1, s0, s1):
    c = lax.axis_index("c")
    s = lax.axis_index("s")
    wid = c * NS + s

    stripe = pl.ds(s * ROWS_PER_TILE, ROWS_PER_TILE)
    pltpu.sync_copy(zeros_hbm.at[stripe], acc_s.at[stripe])
    plsc.subcore_barrier()

    def scale(b, k):
        def group_body(g, carry2):
            wv = w_v[k, pl.ds(g * 16, 16)]
            for t in range(16):
                i = g * 16 + t
                w = wv[t]
                for j in range(D_IN // 16):
                    sl = pl.ds(j * 16, 16)
                    rows_v[b, i, sl] = rows_v[b, i, sl] * w
            return carry2

        lax.fori_loop(0, CHUNK // 16, group_body, 0)

    @pl.loop(0, NQ)
    def quarter_body(qh):
        pltpu.sync_copy(src_hbm.at[wid, qh], src_v)
        pltpu.sync_copy(dst_hbm.at[wid, qh], dst_v)
        pltpu.sync_copy(w_hbm.at[wid, qh], w_v)

        pltpu.async_copy(x_hbm.at[src_v.at[0]], rows_v.at[0], g0)

        @pl.loop(0, H, step=2)
        def chunk_round(k0):
            k1 = k0 + 1
            pltpu.make_async_copy(x_hbm.at[src_v.at[k0]], rows_v.at[0],
                                  g0).wait()
            @pl.when(k0 > 0)
            def _():
                pltpu.make_async_copy(rows_v.at[1],
                                      acc_s.at[dst_v.at[k0 - 1]],
                                      s1).wait()
            pltpu.async_copy(x_hbm.at[src_v.at[k1]], rows_v.at[1], g1)
            scale(0, k0)
            pltpu.async_copy(rows_v.at[0], acc_s.at[dst_v.at[k0]], s0,
                             add=True)
            pltpu.make_async_copy(x_hbm.at[src_v.at[k1]], rows_v.at[1],
                                  g1).wait()
            pltpu.make_async_copy(rows_v.at[0], acc_s.at[dst_v.at[k0]],
                                  s0).wait()
            @pl.when(k1 < H - 1)
            def _():
                pltpu.async_copy(x_hbm.at[src_v.at[k0 + 2]], rows_v.at[0],
                                 g0)
            scale(1, k1)
            pltpu.async_copy(rows_v.at[1], acc_s.at[dst_v.at[k1]], s1,
                             add=True)

        pltpu.make_async_copy(rows_v.at[1], acc_s.at[dst_v.at[H - 1]],
                              s1).wait()

    plsc.subcore_barrier()

    pltpu.sync_copy(acc_s.at[stripe], part_hbm.at[c, stripe])


R_BLK = 1000


def _dense_body(p0_ref, p1_ref, x_ref, wrel_ref, brel_ref, wroot_ref,
                wdec_ref, bdec_ref, out_ref):
    agg = p0_ref[...] + p1_ref[...]
    z = (jnp.dot(agg, wrel_ref[...], preferred_element_type=jnp.float32)
         + jnp.dot(x_ref[...], wroot_ref[...], preferred_element_type=jnp.float32)
         + brel_ref[...])
    z = jnp.maximum(z, 0.0)
    out_ref[...] = (jnp.dot(z, wdec_ref[...], preferred_element_type=jnp.float32)
                    + bdec_ref[...])


def _dense(p0, p1, x, W_rel, b_rel2, W_root, W_dec, b_dec2):
    row = lambda r, cc: pl.BlockSpec((r, cc), lambda i: (i, 0))
    full = lambda a, b: pl.BlockSpec((a, b), lambda i: (0, 0))
    return pl.pallas_call(
        _dense_body,
        grid=(N // R_BLK,),
        in_specs=[row(R_BLK, D_IN), row(R_BLK, D_IN), row(R_BLK, D_IN),
                  full(D_IN, D_OUT), full(1, D_OUT), full(D_IN, D_OUT),
                  full(D_OUT, D_IN), full(1, D_IN)],
        out_specs=row(R_BLK, D_IN),
        out_shape=jax.ShapeDtypeStruct((N, D_IN), jnp.float32),
    )(p0, p1, x, W_rel, b_rel2, W_root, W_dec, b_dec2)


def kernel(x, edge_index, edge_weight, W_rel, b_rel, W_root, W_dec, b_dec):
    src = edge_index[0]
    dst = edge_index[1]
    pad = E_PAD - E
    src_r = jnp.pad(src, (0, pad)).reshape(NW, NQ, H, CHUNK)
    dst_r = jnp.pad(dst, (0, pad)).reshape(NW, NQ, H, CHUNK)
    w_r = jnp.pad(edge_weight, (0, pad)).reshape(NW, NQ, H, CHUNK)
    zeros = jnp.zeros((N_PAD, D_IN), jnp.float32)

    parts = _sc_message_pass(x, src_r, dst_r, w_r, zeros)
    return _dense(parts[0, :N], parts[1, :N], x, W_rel, b_rel.reshape(1, D_OUT),
                  W_root, W_dec, b_dec.reshape(1, D_IN))
